# Initial kernel scaffold; baseline (speedup 1.0000x reference)
import jax
import jax.numpy as jnp
from jax.experimental import pallas as pl


def kernel(gene_x, disease_x, edge_gg, edge_dd, edge_dg, edge_gd, pos_edge, neg_edge, params):
    raise NotImplementedError("write your pallas kernel here")


# speedup vs baseline: 5.9509x; 5.9509x over previous
import functools

import jax
import jax.numpy as jnp
from jax import lax
from jax.experimental import pallas as pl
from jax.experimental.pallas import tpu as pltpu
from jax.experimental.pallas import tpu_sc as plsc

NGN, NDN = 10000, 5000
GFD, DFD = 128, 128
HIDD, OUTD = 256, 128
EGGN, EDDN, EDGN, EGDN = 320000, 80000, 160000, 160000
NPOSN, NNEGN = 4096, 4096

NC, NS, L = 2, 16, 16
NW = NC * NS
CH = 128

NGP = 10000
NDP = 5008
NG4 = 10240
ND4 = 5120
WCH = 64

_MESH = plsc.VectorSubcoreMesh(core_axis_name="c", subcore_axis_name="s",
                               num_cores=NC, num_subcores=NS)


def _pad_to(x, n, val=0):
    return jnp.concatenate([x, jnp.full((n - x.shape[0],) + x.shape[1:], val, x.dtype)])


def _wid():
    return lax.axis_index("s") * NC + lax.axis_index("c")


_GDN = lax.GatherDimensionNumbers(offset_dims=(), collapsed_slice_dims=(0,),
                                  start_index_map=(0,))


def _vgather(v, idx):
    return lax.gather(v, idx[:, None], _GDN, (1,),
                      mode=lax.GatherScatterMode.PROMISE_IN_BOUNDS)


def _splat_sum(v):
    lanes = lax.iota(jnp.int32, L)
    for sh in (1, 2, 4, 8):
        v = v + _vgather(v, lanes ^ sh)
    return v


@functools.lru_cache(maxsize=None)
def _sc_scalar_scatter(e_pad, n_out):
    per_w = e_pad // NW

    @functools.partial(
        pl.kernel,
        out_type=jax.ShapeDtypeStruct((NW, n_out), jnp.float32),
        mesh=_MESH,
        compiler_params=pltpu.CompilerParams(needs_layout_passes=False),
        scratch_types=[
            pltpu.VMEM((n_out,), jnp.float32),
            pltpu.VMEM((per_w,), jnp.float32),
            pltpu.VMEM((per_w,), jnp.int32),
        ],
    )
    def k(vals_hbm, dst_hbm, out_hbm, acc_v, vals_v, dst_v):
        w = _wid()
        base = w * per_w
        pltpu.sync_copy(vals_hbm.at[pl.ds(base, per_w)], vals_v)
        pltpu.sync_copy(dst_hbm.at[pl.ds(base, per_w)], dst_v)

        def zero(i, _):
            acc_v[pl.ds(i * L, L)] = jnp.zeros((L,), jnp.float32)
            return 0
        lax.fori_loop(0, n_out // L, zero, 0)

        def body(i, _):
            d = dst_v[pl.ds(i * L, L)]
            v = vals_v[pl.ds(i * L, L)]
            plsc.addupdate_scatter(acc_v, [d], v)
            return 0
        lax.fori_loop(0, per_w // L, body, 0)
        pltpu.sync_copy(acc_v, out_hbm.at[w])

    return k


@functools.lru_cache(maxsize=None)
def _sc_edge_norm(e_pad, n_tab):
    per_w = e_pad // NW

    @functools.partial(
        pl.kernel,
        out_type=jax.ShapeDtypeStruct((e_pad,), jnp.float32),
        mesh=_MESH,
        compiler_params=pltpu.CompilerParams(needs_layout_passes=False),
        scratch_types=[
            pltpu.VMEM((n_tab,), jnp.float32),
            pltpu.VMEM((per_w,), jnp.int32),
            pltpu.VMEM((per_w,), jnp.int32),
            pltpu.VMEM((per_w,), jnp.float32),
        ],
    )
    def k(dinv_hbm, src_hbm, dst_hbm, out_hbm, tab_v, src_v, dst_v, w_v):
        w = _wid()
        base = w * per_w
        pltpu.sync_copy(dinv_hbm, tab_v)
        pltpu.sync_copy(src_hbm.at[pl.ds(base, per_w)], src_v)
        pltpu.sync_copy(dst_hbm.at[pl.ds(base, per_w)], dst_v)

        def body(i, _):
            s = src_v[pl.ds(i * L, L)]
            d = dst_v[pl.ds(i * L, L)]
            a = plsc.load_gather(tab_v, [s])
            b = plsc.load_gather(tab_v, [d])
            w_v[pl.ds(i * L, L)] = a * b
            return 0
        lax.fori_loop(0, per_w // L, body, 0)
        pltpu.sync_copy(w_v, out_hbm.at[pl.ds(base, per_w)])

    return k


@functools.lru_cache(maxsize=None)
def _sc_edge_alpha(e_pad, n_tab):
    per_w = e_pad // NW

    @functools.partial(
        pl.kernel,
        out_type=jax.ShapeDtypeStruct((e_pad,), jnp.float32),
        mesh=_MESH,
        compiler_params=pltpu.CompilerParams(needs_layout_passes=False),
        scratch_types=[
            pltpu.VMEM((n_tab,), jnp.float32),
            pltpu.VMEM((per_w,), jnp.float32),
            pltpu.VMEM((per_w,), jnp.int32),
            pltpu.VMEM((per_w,), jnp.float32),
        ],
    )
    def k(rden_hbm, ex_hbm, dst_hbm, out_hbm, tab_v, ex_v, dst_v, w_v):
        w = _wid()
        base = w * per_w
        pltpu.sync_copy(rden_hbm, tab_v)
        pltpu.sync_copy(ex_hbm.at[pl.ds(base, per_w)], ex_v)
        pltpu.sync_copy(dst_hbm.at[pl.ds(base, per_w)], dst_v)

        def body(i, _):
            d = dst_v[pl.ds(i * L, L)]
            b = plsc.load_gather(tab_v, [d])
            w_v[pl.ds(i * L, L)] = ex_v[pl.ds(i * L, L)] * b
            return 0
        lax.fori_loop(0, per_w // L, body, 0)
        pltpu.sync_copy(w_v, out_hbm.at[pl.ds(base, per_w)])

    return k


@functools.lru_cache(maxsize=None)
def _sc_gat_ex(e_pad, n_src, n_dst):
    per_w = e_pad // NW
    n_ch = per_w // CH

    @functools.partial(
        pl.kernel,
        out_type=jax.ShapeDtypeStruct((e_pad,), jnp.float32),
        mesh=_MESH,
        compiler_params=pltpu.CompilerParams(needs_layout_passes=False),
        scratch_types=[
            pltpu.VMEM((per_w,), jnp.int32),
            pltpu.VMEM((per_w,), jnp.int32),
            pltpu.VMEM((per_w,), jnp.float32),
            pltpu.VMEM((OUTD,), jnp.float32),
            pltpu.VMEM((CH, OUTD), jnp.float32),
            pltpu.VMEM((CH, OUTD), jnp.float32),
            pltpu.SemaphoreType.DMA,
            pltpu.SemaphoreType.DMA,
        ],
    )
    def k(hl_hbm, hr_hbm, att_hbm, src_hbm, dst_hbm, out_hbm,
          src_v, dst_v, ex_v, att_v, rl_v, rr_v, sem1, sem2):
        w = _wid()
        base = w * per_w
        pltpu.sync_copy(att_hbm, att_v)
        pltpu.sync_copy(src_hbm.at[pl.ds(base, per_w)], src_v)
        pltpu.sync_copy(dst_hbm.at[pl.ds(base, per_w)], dst_v)

        def chunk(c, _):
            cp1 = pltpu.async_copy(hl_hbm.at[src_v.at[pl.ds(c * CH, CH)]], rl_v, sem1)
            cp2 = pltpu.async_copy(hr_hbm.at[dst_v.at[pl.ds(c * CH, CH)]], rr_v, sem2)
            cp1.wait()
            cp2.wait()

            def edge(j, _):
                acc = jnp.zeros((L,), jnp.float32)
                for cc in range(OUTD // L):
                    u = rl_v[j, pl.ds(cc * L, L)] + rr_v[j, pl.ds(cc * L, L)]
                    u = jnp.where(u >= 0.0, u, 0.2 * u)
                    acc = acc + u * att_v[pl.ds(cc * L, L)]
                ev = jnp.exp(_splat_sum(acc))
                idx = jnp.zeros((L,), jnp.int32) + (c * CH + j)
                plsc.store_scatter(ex_v, [idx], ev)
                return 0
            lax.fori_loop(0, CH, edge, 0)
            return 0
        lax.fori_loop(0, n_ch, chunk, 0)
        pltpu.sync_copy(ex_v, out_hbm.at[pl.ds(base, per_w)])

    return k


@functools.lru_cache(maxsize=None)
def _sc_seg_rows(e_pad, n_tab, n_acc, split, off):
    per_w = e_pad // NW
    n_ch = per_w // CH
    rpt = n_acc // NS
    n_wo = rpt // WCH

    @functools.partial(
        pl.kernel,
        out_type=jax.ShapeDtypeStruct((NC, n_acc, OUTD), jnp.float32),
        mesh=_MESH,
        compiler_params=pltpu.CompilerParams(needs_layout_passes=False),
        scratch_types=[
            pltpu.VMEM_SHARED((n_acc, OUTD), jnp.float32),
            pltpu.VMEM((per_w,), jnp.int32),
            pltpu.VMEM((per_w,), jnp.float32),
            pltpu.VMEM((CH,), jnp.int32),
            pltpu.VMEM((CH,), jnp.int32),
            pltpu.VMEM((CH, OUTD), jnp.float32),
            pltpu.SemaphoreType.DMA,
        ],
    )
    def k(h_hbm, src_hbm, dst_hbm, w_hbm, out_hbm,
          acc_sh, src_v, w_v, srci_v, dsti_v, rows_v, sem):
        cid = lax.axis_index("c")
        sid = lax.axis_index("s")
        w = sid * NC + cid
        base = w * per_w
        pltpu.sync_copy(src_hbm.at[pl.ds(base, per_w)], src_v)
        pltpu.sync_copy(w_hbm.at[pl.ds(base, per_w)], w_v)

        def zrow(j, _):
            for cc in range(OUTD // L):
                rows_v[j, pl.ds(cc * L, L)] = jnp.zeros((L,), jnp.float32)
            return 0
        lax.fori_loop(0, CH, zrow, 0)

        def zacc(t, _):
            pltpu.sync_copy(rows_v.at[pl.ds(0, WCH)],
                            acc_sh.at[pl.ds(sid * rpt + t * WCH, WCH)])
            return 0
        lax.fori_loop(0, n_wo, zacc, 0)
        plsc.subcore_barrier()

        def chunk(c, _):
            def adj(j2, _):
                s = src_v[pl.ds(c * CH + j2 * L, L)]
                eid = (jnp.zeros((L,), jnp.int32) + (base + c * CH + j2 * L)
                       + lax.iota(jnp.int32, L))
                s = jnp.where(eid >= split, s + off, s)
                srci_v[pl.ds(j2 * L, L)] = s
                return 0
            lax.fori_loop(0, CH // L, adj, 0)
            pltpu.async_copy(h_hbm.at[srci_v], rows_v, sem).wait()

            def scale(j, _):
                wsp = plsc.load_gather(w_v, [jnp.zeros((L,), jnp.int32) + (c * CH + j)])
                for cc in range(OUTD // L):
                    rows_v[j, pl.ds(cc * L, L)] = rows_v[j, pl.ds(cc * L, L)] * wsp
                return 0
            lax.fori_loop(0, CH, scale, 0)

            pltpu.sync_copy(dst_hbm.at[pl.ds(base + c * CH, CH)], dsti_v)
            pltpu.sync_copy(rows_v, acc_sh.at[dsti_v], add=True)
            return 0
        lax.fori_loop(0, n_ch, chunk, 0)
        plsc.subcore_barrier()

        def wo(t, _):
            r0 = sid * rpt + t * WCH
            pltpu.sync_copy(acc_sh.at[pl.ds(r0, WCH)], rows_v.at[pl.ds(0, WCH)])
            pltpu.sync_copy(rows_v.at[pl.ds(0, WCH)], out_hbm.at[cid, pl.ds(r0, WCH)])
            return 0
        lax.fori_loop(0, n_wo, wo, 0)

    return k


@functools.lru_cache(maxsize=None)
def _sc_pair_gather(n_idx, n_tab, split, off):
    per_w = n_idx // NW
    n_ch = per_w // CH

    @functools.partial(
        pl.kernel,
        out_type=jax.ShapeDtypeStruct((n_idx, OUTD), jnp.float32),
        mesh=_MESH,
        compiler_params=pltpu.CompilerParams(needs_layout_passes=False),
        scratch_types=[
            pltpu.VMEM((per_w,), jnp.int32),
            pltpu.VMEM((CH,), jnp.int32),
            pltpu.VMEM((CH, OUTD), jnp.float32),
            pltpu.SemaphoreType.DMA,
        ],
    )
    def k(tab_hbm, idx_hbm, out_hbm, idx_v, idxo_v, rows_v, sem):
        w = _wid()
        base = w * per_w
        pltpu.sync_copy(idx_hbm.at[pl.ds(base, per_w)], idx_v)

        def chunk(c, _):
            def adj(j2, _):
                s = idx_v[pl.ds(c * CH + j2 * L, L)]
                eid = (jnp.zeros((L,), jnp.int32) + (base + c * CH + j2 * L)
                       + lax.iota(jnp.int32, L))
                s = jnp.where(eid >= split, s + off, s)
                idxo_v[pl.ds(j2 * L, L)] = s
                return 0
            lax.fori_loop(0, CH // L, adj, 0)
            pltpu.async_copy(tab_hbm.at[idxo_v], rows_v, sem).wait()
            pltpu.sync_copy(rows_v, out_hbm.at[pl.ds(base + c * CH, CH)])
            return 0
        lax.fori_loop(0, n_ch, chunk, 0)

    return k


def _mm(x, W, b, bn=None):
    n, kd = x.shape
    m = W.shape[1]
    if bn is None:
        bn = 1024 if n % 1024 == 0 else (1000 if n % 1000 == 0 else n)
    b2 = b.reshape(1, m)

    def body(x_ref, w_ref, b_ref, o_ref):
        o_ref[...] = (jnp.dot(x_ref[...], w_ref[...],
                              preferred_element_type=jnp.float32) + b_ref[...])

    return pl.pallas_call(
        body,
        grid=(n // bn,),
        in_specs=[
            pl.BlockSpec((bn, kd), lambda i: (i, 0)),
            pl.BlockSpec((kd, m), lambda i: (0, 0)),
            pl.BlockSpec((1, m), lambda i: (0, 0)),
        ],
        out_specs=pl.BlockSpec((bn, m), lambda i: (i, 0)),
        out_shape=jax.ShapeDtypeStruct((n, m), jnp.float32),
    )(x, W, b2)


def _ln_relu(y, g, b):
    mmean = jnp.mean(y, axis=-1, keepdims=True)
    var = jnp.mean((y - mmean) ** 2, axis=-1, keepdims=True)
    return jax.nn.relu((y - mmean) * lax.rsqrt(var + 1e-5) * g + b)


def _gating(x, p, ref_d, rel_d):
    n = x.shape[0]
    bn = 1000
    Wg1 = p['Wg'][:HIDD]
    Wg2 = p['Wg'][HIDD:2 * HIDD]
    Wg3 = p['Wg'][2 * HIDD:]

    def body(x_ref, W1, b1, g1, be1, W2, b2, g2, be2, Wv, bv, Wo, bo,
             Wg1r, Wg2r, Wg3r, bg, o_ref):
        xb = x_ref[...]
        ref_e = _ln_relu(jnp.dot(xb[:, :ref_d], W1[...],
                                 preferred_element_type=jnp.float32) + b1[...],
                         g1[...], be1[...])
        rel_e = _ln_relu(jnp.dot(xb[:, ref_d:], W2[...],
                                 preferred_element_type=jnp.float32) + b2[...],
                         g2[...], be2[...])
        v = jnp.dot(rel_e, Wv[...], preferred_element_type=jnp.float32) + bv[...]
        attn_out = jnp.dot(v, Wo[...], preferred_element_type=jnp.float32) + bo[...]
        z = (jnp.dot(ref_e, Wg1r[...], preferred_element_type=jnp.float32)
             + jnp.dot(rel_e, Wg2r[...], preferred_element_type=jnp.float32)
             + jnp.dot(attn_out, Wg3r[...], preferred_element_type=jnp.float32)
             + bg[...])
        gate = jax.nn.sigmoid(z)
        o_ref[...] = gate * ref_e + (1.0 - gate) * rel_e

    row = lambda a: a.reshape(1, -1)
    full = lambda shp: pl.BlockSpec(shp, lambda i: (0, 0))
    return pl.pallas_call(
        body,
        grid=(n // bn,),
        in_specs=[pl.BlockSpec((bn, ref_d + rel_d), lambda i: (i, 0)),
                  full((ref_d, HIDD)), full((1, HIDD)), full((1, HIDD)), full((1, HIDD)),
                  full((rel_d, HIDD)), full((1, HIDD)), full((1, HIDD)), full((1, HIDD)),
                  full((HIDD, HIDD)), full((1, HIDD)),
                  full((HIDD, HIDD)), full((1, HIDD)),
                  full((HIDD, 1)), full((HIDD, 1)), full((HIDD, 1)), full((1, 1))],
        out_specs=pl.BlockSpec((bn, HIDD), lambda i: (i, 0)),
        out_shape=jax.ShapeDtypeStruct((n, HIDD), jnp.float32),
    )(x, p['W1'], row(p['b1']), row(p['g1']), row(p['be1']),
      p['W2'], row(p['b2']), row(p['g2']), row(p['be2']),
      p['Wv'], row(p['bv']), p['Wo'], row(p['bo']),
      Wg1, Wg2, Wg3, p['bg'].reshape(1, 1))


def _dinv_of_partials(partials):
    nw, n = partials.shape

    def body(p_ref, d_ref, d2_ref):
        deg = jnp.sum(p_ref[...], axis=0, keepdims=True) + 1.0
        dinv = lax.rsqrt(deg)
        d_ref[...] = dinv
        d2_ref[...] = dinv * dinv

    return pl.pallas_call(
        body,
        out_shape=[jax.ShapeDtypeStruct((1, n), jnp.float32),
                   jax.ShapeDtypeStruct((1, n), jnp.float32)],
    )(partials)


def _rden_of_partials(partials):
    nw, n = partials.shape

    def body(p_ref, o_ref):
        den = jnp.sum(p_ref[...], axis=0, keepdims=True)
        o_ref[...] = 1.0 / (den + 1e-16)

    return pl.pallas_call(
        body,
        out_shape=jax.ShapeDtypeStruct((1, n), jnp.float32),
    )(partials)


def _combine_bn_se(p0, p1, bias, bn_g, bn_b, se1, se2, res=None):
    n = p0.shape[0]
    ins = [p0, p1, bias.reshape(1, OUTD), bn_g.reshape(1, OUTD),
           bn_b.reshape(1, OUTD), se1, se2]
    if res is not None:
        ins.append(res)

    def body(*refs):
        if res is not None:
            p0r, p1r, br, gr, bbr, s1r, s2r, rr, o_ref = refs
        else:
            p0r, p1r, br, gr, bbr, s1r, s2r, o_ref = refs
        x = p0r[...] + p1r[...] + br[...]
        m = jnp.mean(x, axis=0, keepdims=True)
        v = jnp.mean((x - m) ** 2, axis=0, keepdims=True)
        x = jax.nn.relu((x - m) * lax.rsqrt(v + 1e-5) * gr[...] + bbr[...])
        y = jax.nn.sigmoid(
            jnp.dot(jax.nn.relu(jnp.dot(jnp.mean(x, axis=0, keepdims=True), s1r[...],
                                        preferred_element_type=jnp.float32)),
                    s2r[...], preferred_element_type=jnp.float32))
        x = x * y
        if res is not None:
            x = x + rr[...]
        o_ref[...] = x

    return pl.pallas_call(
        body,
        out_shape=jax.ShapeDtypeStruct((n, OUTD), jnp.float32),
    )(*ins)


def _final_mlp(conbs, Wm1, bm1, Wm2, bm2):
    n = conbs.shape[0]

    def body(x_ref, w1, b1, w2, b2, probs_ref, loss_ref):
        h = jax.nn.relu(jnp.dot(x_ref[...], w1[...],
                                preferred_element_type=jnp.float32) + b1[...])
        z = jnp.dot(h, w2[...], preferred_element_type=jnp.float32) + b2[...]
        probs = jax.nn.sigmoid(z)
        probs_ref[...] = probs
        pc = jnp.clip(probs, 1e-7, 1.0 - 1e-7)
        tgt = (lax.broadcasted_iota(jnp.int32, (n, 1), 0) < NPOSN).astype(jnp.float32)
        ll = tgt * jnp.log(pc) + (1.0 - tgt) * jnp.log(1.0 - pc)
        loss_ref[...] = -jnp.mean(ll, keepdims=True)

    return pl.pallas_call(
        body,
        out_shape=[jax.ShapeDtypeStruct((n, 1), jnp.float32),
                   jax.ShapeDtypeStruct((1, 1), jnp.float32)],
    )(conbs, Wm1, bm1.reshape(1, OUTD), Wm2, bm2.reshape(1, 1))


def _ceil_pad(e):
    blk = NW * CH
    return ((e + blk - 1) // blk) * blk


def _gat_layer_ex(hl, hr, att, src, dst, e_real):
    e_pad = _ceil_pad(e_real)
    srcp = _pad_to(src, e_pad)
    dstp = _pad_to(dst, e_pad)
    ex = _sc_gat_ex(e_pad, hl.shape[0], hr.shape[0])(hl, hr, att, srcp, dstp)
    return ex[:e_real], dstp


def kernel(gene_x, disease_x, edge_gg, edge_dd, edge_dg, edge_gd,
           pos_edge, neg_edge, params):
    p = params
    edge_gg = edge_gg.astype(jnp.int32)
    edge_dd = edge_dd.astype(jnp.int32)
    edge_dg = edge_dg.astype(jnp.int32)
    edge_gd = edge_gd.astype(jnp.int32)

    x_g = _gating(gene_x, p['g_gate'], GFD, HIDD)
    x_d = _gating(disease_x, p['d_gate'], DFD, HIDD)
    res_g = _mm(x_g, p['Wgl'], p['bgl'])
    res_d = _mm(x_d, p['Wdl'], p['bdl'])

    egg_pad = _ceil_pad(EGGN)
    edd_pad = _ceil_pad(EDDN)
    gg_s = _pad_to(edge_gg[0], egg_pad)
    gg_d = _pad_to(edge_gg[1], egg_pad)
    dd_s = _pad_to(edge_dd[0], edd_pad)
    dd_d = _pad_to(edge_dd[1], edd_pad)
    ones_gg = _pad_to(jnp.ones((EGGN,), jnp.float32), egg_pad)
    ones_dd = _pad_to(jnp.ones((EDDN,), jnp.float32), edd_pad)

    degp_g = _sc_scalar_scatter(egg_pad, NGP)(ones_gg, gg_d)
    degp_d = _sc_scalar_scatter(edd_pad, NDP)(ones_dd, dd_d)
    dinv_g, dinv2_g = _dinv_of_partials(degp_g)
    dinv_d, dinv2_d = _dinv_of_partials(degp_d)
    dinv_g, dinv2_g = dinv_g[0], dinv2_g[0]
    dinv_d, dinv2_d = dinv_d[0], dinv2_d[0]

    norm_gg = _sc_edge_norm(egg_pad, NGP)(dinv_g, gg_s, gg_d)[:EGGN]
    norm_dd = _sc_edge_norm(edd_pad, NDP)(dinv_d, dd_s, dd_d)[:EDDN]

    ar_g = jnp.arange(NGN, dtype=jnp.int32)
    ar_d = jnp.arange(NDN, dtype=jnp.int32)

    gsrc = jnp.concatenate([edge_gg[0], ar_g, edge_dg[0]])
    gdst = jnp.concatenate([edge_gg[1], ar_g, edge_dg[1]])
    eg_real = EGGN + NGN + EDGN
    eg_pad = _ceil_pad(eg_real)
    gsrc = _pad_to(gsrc, eg_pad)
    gdst = _pad_to(gdst, eg_pad)

    dsrc = jnp.concatenate([edge_dd[0], ar_d, edge_gd[0]])
    ddst = jnp.concatenate([edge_dd[1], ar_d, edge_gd[1]])
    ed_real = EDDN + NDN + EGDN
    ed_pad = _ceil_pad(ed_real)
    dsrc = _pad_to(dsrc, ed_pad)
    ddst = _pad_to(ddst, ed_pad)

    for li, lp in enumerate(p['layers']):
        gp = lp['gat_dg']
        gq = lp['gat_gd']
        Wg_cat = jnp.concatenate([lp['Wgg'], gq['Wl'], gp['Wr']], axis=1)
        bg_cat = jnp.concatenate([jnp.zeros_like(lp['bgg']), gq['bl'], gp['br']])
        Wd_cat = jnp.concatenate([lp['Wdd'], gp['Wl'], gq['Wr']], axis=1)
        bd_cat = jnp.concatenate([jnp.zeros_like(lp['bdd']), gp['bl'], gq['br']])
        hg3 = _mm(x_g, Wg_cat, bg_cat)
        hd3 = _mm(x_d, Wd_cat, bd_cat)
        h_gg, hl_gd, hr_dg = hg3[:, :OUTD], hg3[:, OUTD:2 * OUTD], hg3[:, 2 * OUTD:]
        h_dd, hl_dg, hr_gd = hd3[:, :OUTD], hd3[:, OUTD:2 * OUTD], hd3[:, 2 * OUTD:]

        ex_dg, dst_dg_p = _gat_layer_ex(hl_dg, hr_dg, gp['att'], edge_dg[0],
                                        edge_dg[1], EDGN)
        ex_gd, dst_gd_p = _gat_layer_ex(hl_gd, hr_gd, gq['att'], edge_gd[0],
                                        edge_gd[1], EGDN)

        edg_pad = _ceil_pad(EDGN)
        egd_pad = _ceil_pad(EGDN)
        denp_g = _sc_scalar_scatter(edg_pad, NGP)(_pad_to(ex_dg, edg_pad), dst_dg_p)
        denp_d = _sc_scalar_scatter(egd_pad, NDP)(_pad_to(ex_gd, egd_pad), dst_gd_p)
        rden_g = _rden_of_partials(denp_g)[0]
        rden_d = _rden_of_partials(denp_d)[0]

        alpha_dg = _sc_edge_alpha(edg_pad, NGP)(
            rden_g, _pad_to(ex_dg, edg_pad), dst_dg_p)[:EDGN]
        alpha_gd = _sc_edge_alpha(egd_pad, NDP)(
            rden_d, _pad_to(ex_gd, egd_pad), dst_gd_p)[:EGDN]

        Hg = jnp.concatenate([h_gg, hl_dg], axis=0)
        wg = _pad_to(jnp.concatenate([norm_gg, dinv2_g[:NGN], alpha_dg]), eg_pad)
        pg = _sc_seg_rows(eg_pad, Hg.shape[0], NG4, EGGN + NGN, NGN)(
            Hg, gsrc, gdst, wg)

        Hd = jnp.concatenate([h_dd, hl_gd], axis=0)
        wd = _pad_to(jnp.concatenate([norm_dd, dinv2_d[:NDN], alpha_gd]), ed_pad)
        pd = _sc_seg_rows(ed_pad, Hd.shape[0], ND4, EDDN + NDN, NDN)(
            Hd, dsrc, ddst, wd)

        bias_g = lp['bgg'] + gp['bias']
        bias_d = lp['bdd'] + gq['bias']
        last = li == len(p['layers']) - 1
        x_g = _combine_bn_se(pg[0, :NGN], pg[1, :NGN], bias_g, lp['bn_g'],
                             lp['bn_b'], lp['se1'], lp['se2'],
                             res=res_g if last else None)
        x_d = _combine_bn_se(pd[0, :NDN], pd[1, :NDN], bias_d, lp['bn_g'],
                             lp['bn_b'], lp['se1'], lp['se2'],
                             res=res_d if last else None)

    pairs = jnp.concatenate([pos_edge, neg_edge], 0).astype(jnp.int32)
    tab = jnp.concatenate([x_d, x_g], axis=0)
    idx = jnp.concatenate([pairs[:, 0], pairs[:, 1]])
    npair = NPOSN + NNEGN
    rows = _sc_pair_gather(2 * npair, tab.shape[0], npair, NDN)(tab, idx)
    conbs = jnp.concatenate([rows[:npair], rows[npair:]], axis=1)
    probs, loss = _final_mlp(conbs, p['Wm1'], p['bm1'], p['Wm2'], p['bm2'])
    return loss[0, 0], probs[:, 0]
